# Initial kernel scaffold; baseline (speedup 1.0000x reference)
import jax
import jax.numpy as jnp
from jax.experimental import pallas as pl


def kernel(preds, trgs):
    raise NotImplementedError("write your pallas kernel here")


# speedup vs baseline: 1.8262x; 1.8262x over previous
import math

import jax
import jax.numpy as jnp
from jax.experimental import pallas as pl
from jax.experimental.pallas import tpu as pltpu

N = 1024
V = 100000
PAD = 0
EPS = 0.1
CONF = 1.0 - EPS
U = EPS / (V - 2)
C0 = EPS * math.log(U) + CONF * math.log(CONF)

CB = 2048
NBLK = (V + CB - 1) // CB


def _body(trg_ref, x_ref, out_ref, acc_ref, gacc_ref, p0_ref):
    j = pl.program_id(0)
    x = x_ref[...]
    trg = trg_ref[...]
    col = jax.lax.broadcasted_iota(jnp.int32, (N, CB), 1) + j * CB
    valid = col < V
    xv = jnp.where(valid, x, 0.0)
    acc = jnp.sum(xv, axis=1, keepdims=True)
    g = jnp.sum(jnp.where(col == trg, xv, 0.0), axis=1, keepdims=True)

    @pl.when(j == 0)
    def _init():
        acc_ref[...] = acc
        gacc_ref[...] = g
        p0_ref[...] = x[:, 0:1]

    @pl.when(j > 0)
    def _accum():
        acc_ref[...] += acc
        gacc_ref[...] += g

    @pl.when(j == NBLK - 1)
    def _final():
        s = acc_ref[...]
        gg = gacc_ref[...]
        p0 = p0_ref[...]
        term = C0 - U * (s - p0 - gg) - CONF * gg
        term = jnp.where(trg != PAD, term, 0.0)
        out_ref[0, 0] = jnp.sum(term) / N


def kernel(preds, trgs):
    trg2 = trgs.astype(jnp.int32).reshape(N, 1)
    out = pl.pallas_call(
        _body,
        grid=(NBLK,),
        in_specs=[
            pl.BlockSpec((N, 1), lambda j: (0, 0)),
            pl.BlockSpec((N, CB), lambda j: (0, j)),
        ],
        out_specs=pl.BlockSpec((1, 1), lambda j: (0, 0), memory_space=pltpu.SMEM),
        out_shape=jax.ShapeDtypeStruct((1, 1), jnp.float32),
        scratch_shapes=[
            pltpu.VMEM((N, 1), jnp.float32),
            pltpu.VMEM((N, 1), jnp.float32),
            pltpu.VMEM((N, 1), jnp.float32),
        ],
        compiler_params=pltpu.CompilerParams(
            dimension_semantics=("arbitrary",),
        ),
    )(trg2, preds)
    return out[0, 0]
